# baseline (speedup 1.0000x reference)
import functools

import jax
import jax.numpy as jnp
from jax import lax
from jax.experimental import pallas as pl
from jax.experimental.pallas import tpu as pltpu
from jax.experimental.pallas import tpu_sc as plsc

N_NODES = 10000
IN_DIM = 128
OUT_DIM = 128
N_REL = 8

_TN = 1000
_K = 128
_NC = 2
_NS = 16
_NW = _NC * _NS
_RPT_Z = 624
_ACC_ROWS = 10016
_Z_TAIL = _ACC_ROWS - _RPT_Z * _NS
_RPT_O = 624
_O_TAIL = N_NODES - _RPT_O * _NS


def _mm_body(x_ref, w_ref, b_ref, o_ref):
    acc = jnp.dot(x_ref[...], w_ref[0], preferred_element_type=jnp.float32)
    o_ref[0] = jnp.maximum(acc + b_ref[0][None, :], 0.0)


def _relu_xw(x, weight, bias):
    n = x.shape[0]
    nt = n // _TN
    return pl.pallas_call(
        _mm_body,
        grid=(N_REL, nt),
        in_specs=[
            pl.BlockSpec((_TN, IN_DIM), lambda r, i: (i, 0)),
            pl.BlockSpec((1, IN_DIM, OUT_DIM), lambda r, i: (r, 0, 0)),
            pl.BlockSpec((1, OUT_DIM), lambda r, i: (0, 0)),
        ],
        out_specs=pl.BlockSpec((1, _TN, OUT_DIM), lambda r, i: (r, i, 0)),
        out_shape=jax.ShapeDtypeStruct((N_REL, n, OUT_DIM), jnp.float32),
    )(x, weight, bias.reshape(1, OUT_DIM))


def _sc_edge_scatter(h, rows, dsts, n_chunks):
    mesh = plsc.VectorSubcoreMesh(core_axis_name="c", subcore_axis_name="s")

    @functools.partial(
        pl.kernel,
        mesh=mesh,
        out_type=jax.ShapeDtypeStruct((_NC, N_NODES, OUT_DIM), jnp.float32),
        scratch_types=[
            pltpu.VMEM_SHARED((_ACC_ROWS, OUT_DIM), jnp.float32),
            pltpu.VMEM((2, 2, _K), jnp.int32),
            pltpu.VMEM((_K, OUT_DIM), jnp.float32),
            pltpu.SemaphoreType.DMA,
            pltpu.SemaphoreType.DMA,
            pltpu.SemaphoreType.DMA,
        ],
    )
    def k(pair_hbm, h_hbm, out_hbm, acc, pair_v, rows_v, sem, sp0, sp1):
        c = lax.axis_index("c")
        s = lax.axis_index("s")
        wid = s * _NC + c

        def zrow(j, carry):
            def zcol(q, carry2):
                rows_v[j, pl.ds(q * 16, 16)] = jnp.zeros((16,), jnp.float32)
                return carry2
            return lax.fori_loop(0, OUT_DIM // 16, zcol, carry)
        lax.fori_loop(0, _K, zrow, 0)
        zbase = pl.multiple_of(s * _RPT_Z, 8)
        for t in range(_RPT_Z // _K):
            pltpu.sync_copy(rows_v, acc.at[pl.ds(zbase + t * _K, _K)])
        rem = _RPT_Z % _K
        if rem:
            pltpu.sync_copy(rows_v.at[pl.ds(0, rem)],
                            acc.at[pl.ds(zbase + (_RPT_Z // _K) * _K, rem)])

        @pl.when(s == _NS - 1)
        def _ztail():
            pltpu.sync_copy(rows_v.at[pl.ds(0, _Z_TAIL)],
                            acc.at[pl.ds(_RPT_Z * _NS, _Z_TAIL)])
        plsc.subcore_barrier()

        ebase = wid * n_chunks
        sp = (sp0, sp1)
        pltpu.async_copy(pair_hbm.at[ebase], pair_v.at[0], sp[0])

        def body(j, carry):
            for b in range(2):
                i = j * 2 + b
                pltpu.make_async_copy(pair_hbm.at[0], pair_v.at[b],
                                      sp[b]).wait()

                @pl.when(i + 1 < n_chunks)
                def _prefetch():
                    pltpu.async_copy(pair_hbm.at[ebase + i + 1],
                                     pair_v.at[1 - b], sp[1 - b])

                pltpu.async_copy(h_hbm.at[pair_v.at[b, 0]], rows_v, sem).wait()
                pltpu.sync_copy(rows_v, acc.at[pair_v.at[b, 1]], add=True)
            return carry
        lax.fori_loop(0, n_chunks // 2, body, 0)
        plsc.subcore_barrier()

        ob = pl.multiple_of(s * _RPT_O, 8)
        pltpu.sync_copy(acc.at[pl.ds(ob, _RPT_O)], out_hbm.at[c, pl.ds(ob, _RPT_O)])

        @pl.when(s == _NS - 1)
        def _tail():
            tb = _RPT_O * _NS
            pltpu.sync_copy(acc.at[pl.ds(tb, _O_TAIL)],
                            out_hbm.at[c, pl.ds(tb, _O_TAIL)])

    return k(jnp.stack([rows.reshape(-1, _K), dsts.reshape(-1, _K)], axis=1), h)


def _combine_body(p_ref, o_ref):
    o_ref[...] = p_ref[0] + p_ref[1]


def _combine(parts):
    nt = N_NODES // _TN
    return pl.pallas_call(
        _combine_body,
        grid=(nt,),
        in_specs=[pl.BlockSpec((_NC, _TN, OUT_DIM), lambda i: (0, i, 0))],
        out_specs=pl.BlockSpec((_TN, OUT_DIM), lambda i: (i, 0)),
        out_shape=jax.ShapeDtypeStruct((N_NODES, OUT_DIM), jnp.float32),
    )(parts)


@jax.jit
def kernel(x, edge_index, edge_type, weight, bias):
    n = x.shape[0]
    e = edge_index.shape[1]
    src = edge_index[0].astype(jnp.int32)
    dst = edge_index[1].astype(jnp.int32)
    rel = edge_type.astype(jnp.int32)
    rows = rel * n + src

    n_chunks = -(-e // (_NW * _K * 2)) * 2
    pad = n_chunks * _K * _NW - e
    rows = jnp.concatenate([rows, jnp.zeros((pad,), jnp.int32)])
    dstp = jnp.concatenate([dst, jnp.full((pad,), n, jnp.int32)])

    h = _relu_xw(x, weight, bias).reshape(N_REL * n, OUT_DIM)
    parts = _sc_edge_scatter(h, rows, dstp, n_chunks)
    return _combine(parts)


# speedup vs baseline: 1.3428x; 1.3428x over previous
import functools

import jax
import jax.numpy as jnp
from jax import lax
from jax.experimental import pallas as pl
from jax.experimental.pallas import tpu as pltpu
from jax.experimental.pallas import tpu_sc as plsc

N_NODES = 10000
IN_DIM = 128
OUT_DIM = 128
N_REL = 8

_TN = 1000
_K = 128
_NC = 2
_NS = 16
_NW = _NC * _NS
_RPT_Z = 624
_ACC_ROWS = 10016
_Z_TAIL = _ACC_ROWS - _RPT_Z * _NS
_RPT_O = 624
_O_TAIL = N_NODES - _RPT_O * _NS


def _mm_body(x_ref, w_ref, b_ref, o_ref):
    acc = jnp.dot(x_ref[...], w_ref[0], preferred_element_type=jnp.float32)
    o_ref[0] = jnp.maximum(acc + b_ref[0][None, :], 0.0)


def _relu_xw(x, weight, bias):
    n = x.shape[0]
    nt = n // _TN
    return pl.pallas_call(
        _mm_body,
        grid=(N_REL, nt),
        in_specs=[
            pl.BlockSpec((_TN, IN_DIM), lambda r, i: (i, 0)),
            pl.BlockSpec((1, IN_DIM, OUT_DIM), lambda r, i: (r, 0, 0)),
            pl.BlockSpec((1, OUT_DIM), lambda r, i: (0, 0)),
        ],
        out_specs=pl.BlockSpec((1, _TN, OUT_DIM), lambda r, i: (r, i, 0)),
        out_shape=jax.ShapeDtypeStruct((N_REL, n, OUT_DIM), jnp.float32),
    )(x, weight, bias.reshape(1, OUT_DIM))


def _sc_edge_scatter(h, rows, dsts, n_chunks):
    mesh = plsc.VectorSubcoreMesh(core_axis_name="c", subcore_axis_name="s")

    @functools.partial(
        pl.kernel,
        mesh=mesh,
        out_type=jax.ShapeDtypeStruct((_NC, N_NODES, OUT_DIM), jnp.float32),
        scratch_types=[
            pltpu.VMEM_SHARED((_ACC_ROWS, OUT_DIM), jnp.float32),
            pltpu.VMEM((2, _K), jnp.int32),
            pltpu.VMEM((_K, OUT_DIM), jnp.float32),
            pltpu.SemaphoreType.DMA,
        ],
    )
    def k(pair_hbm, h_hbm, out_hbm, acc, pair_v, rows_v, sem):
        c = lax.axis_index("c")
        s = lax.axis_index("s")
        wid = s * _NC + c

        def zrow(j, carry):
            def zcol(q, carry2):
                rows_v[j, pl.ds(q * 16, 16)] = jnp.zeros((16,), jnp.float32)
                return carry2
            return lax.fori_loop(0, OUT_DIM // 16, zcol, carry)
        lax.fori_loop(0, _K, zrow, 0)
        zbase = pl.multiple_of(s * _RPT_Z, 8)
        for t in range(_RPT_Z // _K):
            pltpu.sync_copy(rows_v, acc.at[pl.ds(zbase + t * _K, _K)])
        rem = _RPT_Z % _K
        if rem:
            pltpu.sync_copy(rows_v.at[pl.ds(0, rem)],
                            acc.at[pl.ds(zbase + (_RPT_Z // _K) * _K, rem)])

        @pl.when(s == _NS - 1)
        def _ztail():
            pltpu.sync_copy(rows_v.at[pl.ds(0, _Z_TAIL)],
                            acc.at[pl.ds(_RPT_Z * _NS, _Z_TAIL)])
        plsc.subcore_barrier()

        ebase = wid * n_chunks

        def body(i, carry):
            pltpu.sync_copy(pair_hbm.at[ebase + i], pair_v)
            pltpu.async_copy(h_hbm.at[pair_v.at[0]], rows_v, sem).wait()
            pltpu.sync_copy(rows_v, acc.at[pair_v.at[1]], add=True)
            return carry
        lax.fori_loop(0, n_chunks, body, 0)
        plsc.subcore_barrier()

        ob = pl.multiple_of(s * _RPT_O, 8)
        pltpu.sync_copy(acc.at[pl.ds(ob, _RPT_O)], out_hbm.at[c, pl.ds(ob, _RPT_O)])

        @pl.when(s == _NS - 1)
        def _tail():
            tb = _RPT_O * _NS
            pltpu.sync_copy(acc.at[pl.ds(tb, _O_TAIL)],
                            out_hbm.at[c, pl.ds(tb, _O_TAIL)])

    return k(jnp.stack([rows.reshape(-1, _K), dsts.reshape(-1, _K)], axis=1), h)


def _combine_body(p_ref, o_ref):
    o_ref[...] = p_ref[0] + p_ref[1]


def _combine(parts):
    nt = N_NODES // _TN
    return pl.pallas_call(
        _combine_body,
        grid=(nt,),
        in_specs=[pl.BlockSpec((_NC, _TN, OUT_DIM), lambda i: (0, i, 0))],
        out_specs=pl.BlockSpec((_TN, OUT_DIM), lambda i: (i, 0)),
        out_shape=jax.ShapeDtypeStruct((N_NODES, OUT_DIM), jnp.float32),
    )(parts)


@jax.jit
def kernel(x, edge_index, edge_type, weight, bias):
    n = x.shape[0]
    e = edge_index.shape[1]
    src = edge_index[0].astype(jnp.int32)
    dst = edge_index[1].astype(jnp.int32)
    rel = edge_type.astype(jnp.int32)
    rows = rel * n + src

    n_chunks = -(-e // (_NW * _K))
    pad = n_chunks * _K * _NW - e
    rows = jnp.concatenate([rows, jnp.zeros((pad,), jnp.int32)])
    dstp = jnp.concatenate([dst, jnp.full((pad,), n, jnp.int32)])

    h = _relu_xw(x, weight, bias).reshape(N_REL * n, OUT_DIM)
    parts = _sc_edge_scatter(h, rows, dstp, n_chunks)
    return _combine(parts)
